# baseline (speedup 1.0000x reference)
import functools

import jax
import jax.numpy as jnp
from jax import lax
from jax.experimental import pallas as pl
from jax.experimental.pallas import tpu as pltpu
from jax.experimental.pallas import tpu_sc as plsc

N_FACTORS = 32
BATCH = 16384
NC = 2
NS = 16
L = 16
NW = NC * NS
B_PER_W = BATCH // NW


def _sc_body(user_hbm, item_hbm, uf_hbm, itf_hbm, ub_hbm, ib_hbm, out_hbm,
             idx_u, idx_i, uf_v, itf_v, ub_v, ib_v, out_v, sem):
    wid = lax.axis_index("s") * NC + lax.axis_index("c")
    base = wid * B_PER_W

    pltpu.sync_copy(user_hbm.at[pl.ds(base, B_PER_W)], idx_u)
    pltpu.sync_copy(item_hbm.at[pl.ds(base, B_PER_W)], idx_i)

    c1 = pltpu.make_async_copy(uf_hbm.at[idx_u], uf_v, sem)
    c2 = pltpu.make_async_copy(itf_hbm.at[idx_i], itf_v, sem)
    c3 = pltpu.make_async_copy(ub_hbm.at[idx_u], ub_v, sem)
    c4 = pltpu.make_async_copy(ib_hbm.at[idx_i], ib_v, sem)
    c1.start()
    c2.start()
    c3.start()
    c4.start()
    c1.wait()
    c2.wait()
    c3.wait()
    c4.wait()

    lane = lax.iota(jnp.int32, L)

    def chunk(c, carry):
        rows = c * L + lane
        acc = ub_v[pl.ds(c * L, L)] + ib_v[pl.ds(c * L, L)]
        for f in range(N_FACTORS):
            col = jnp.full((L,), f, jnp.int32)
            acc = acc + (plsc.load_gather(uf_v, [rows, col]) *
                         plsc.load_gather(itf_v, [rows, col]))
        out_v[pl.ds(c * L, L)] = acc
        return carry

    lax.fori_loop(0, B_PER_W // L, chunk, 0)

    pltpu.sync_copy(out_v, out_hbm.at[pl.ds(base, B_PER_W)])


@jax.jit
def _predict(user, item, user_factors, item_factors, user_biases, item_biases):
    run = pl.kernel(
        _sc_body,
        out_type=jax.ShapeDtypeStruct((BATCH,), jnp.float32),
        mesh=plsc.VectorSubcoreMesh(core_axis_name="c", subcore_axis_name="s"),
        compiler_params=pltpu.CompilerParams(
            needs_layout_passes=False, use_tc_tiling_on_sc=False),
        scratch_types=[
            pltpu.VMEM((B_PER_W,), jnp.int32),
            pltpu.VMEM((B_PER_W,), jnp.int32),
            pltpu.VMEM((B_PER_W, N_FACTORS), jnp.float32),
            pltpu.VMEM((B_PER_W, N_FACTORS), jnp.float32),
            pltpu.VMEM((B_PER_W,), jnp.float32),
            pltpu.VMEM((B_PER_W,), jnp.float32),
            pltpu.VMEM((B_PER_W,), jnp.float32),
            pltpu.SemaphoreType.DMA,
        ],
    )
    return run(user, item, user_factors, item_factors,
               user_biases.reshape(-1), item_biases.reshape(-1))


def kernel(user, item, user_factors, item_factors, user_biases, item_biases):
    return _predict(user, item, user_factors, item_factors,
                    user_biases, item_biases)
